# baseline (speedup 1.0000x reference)
import functools

import jax
import jax.numpy as jnp
from jax import lax
from jax.experimental import pallas as pl
from jax.experimental.pallas import tpu as pltpu
from jax.experimental.pallas import tpu_sc as plsc

D = 32
HIST = 50
HT = 2 * HIST
RW = 2 * D
NT = 100000
NC, NS, L = 2, 16, 16
NW = NC * NS
B = 16384
BT = B // NW
C = 16
NCH = BT // C
CI = C * HT


def _tmf_body(tag_h, rows_h, tbl_h, out_h,
              tag_i, tr, rw, out_v, sem0, sem1):
    wid = lax.axis_index("s") * NC + lax.axis_index("c")
    base = wid * BT

    pltpu.sync_copy(tag_h.at[pl.ds(base * HT, BT * HT)], tag_i)

    def issue(g, k, sem):
        pltpu.async_copy(tbl_h.at[tag_i.at[pl.ds(g * CI, CI)]], tr.at[k], sem)
        pltpu.async_copy(rows_h.at[pl.ds((base + g * C) * RW, C * RW)],
                         rw.at[k], sem)

    def drain(k, sem):
        pltpu.make_async_copy(tbl_h.at[pl.ds(0, CI)], tr.at[k], sem).wait()
        pltpu.make_async_copy(rows_h.at[pl.ds(0, C * RW)], rw.at[k], sem).wait()

    himask = jnp.uint32(0xFFFF0000)

    def _lo(w):
        return plsc.bitcast(w << jnp.uint32(16), jnp.float32)

    def _hi(w):
        return plsc.bitcast(w & himask, jnp.float32)

    def compute(g, k):
        def _row(r):
            return plsc.bitcast(tr[k, r, pl.ds(0, 4 * L)], jnp.uint32)

        def body_b(b, acc):
            r0 = b * HT
            wu = _row(r0)
            wi = _row(r0 + HIST)
            u0, u1 = _lo(wu), _hi(wu)
            i0, i1 = _lo(wi), _hi(wi)
            for h in range(1, HIST):
                wu = _row(r0 + h)
                wi = _row(r0 + HIST + h)
                u0 = u0 + _lo(wu)
                u1 = u1 + _hi(wu)
                i0 = i0 + _lo(wi)
                i1 = i1 + _hi(wi)
            inv = 1.0 / HIST
            ru0 = rw[k, pl.ds(b * RW, L)] + u0 * inv
            ru1 = rw[k, pl.ds(b * RW + L, L)] + u1 * inv
            ri0 = rw[k, pl.ds(b * RW + D, L)] + i0 * inv
            ri1 = rw[k, pl.ds(b * RW + D + L, L)] + i1 * inv
            s = ru0 * ri0 + ru1 * ri1
            iot = lax.iota(jnp.int32, L)
            for k2 in (8, 4, 2, 1):
                s = s + s[jnp.bitwise_xor(iot, k2)]
            return jnp.where(iot == b, s, acc)
        a = lax.fori_loop(0, C, body_b, jnp.zeros((L,), jnp.float32))
        out_v[pl.ds(g * C, C)] = a

    issue(0, 0, sem0)

    def pair(t, carry):
        g0 = 2 * t
        issue(g0 + 1, 1, sem1)
        drain(0, sem0)
        compute(g0, 0)

        @pl.when(t < NCH // 2 - 1)
        def _():
            issue(g0 + 2, 0, sem0)

        drain(1, sem1)
        compute(g0 + 1, 1)
        return carry

    lax.fori_loop(0, NCH // 2, pair, 0)

    pltpu.sync_copy(out_v, out_h.at[pl.ds(base, BT)])


_tmf = functools.partial(
    pl.kernel,
    out_type=jax.ShapeDtypeStruct((B,), jnp.float32),
    mesh=plsc.VectorSubcoreMesh(core_axis_name="c", subcore_axis_name="s",
                                num_cores=NC, num_subcores=NS),
    scratch_types=[
        pltpu.VMEM((BT * HT,), jnp.int32),
        pltpu.VMEM((2, CI, 4 * L), jnp.uint8),
        pltpu.VMEM((2, C * RW), jnp.float32),
        pltpu.VMEM((BT,), jnp.float32),
        pltpu.SemaphoreType.DMA,
        pltpu.SemaphoreType.DMA,
    ],
    compiler_params=pltpu.CompilerParams(use_tc_tiling_on_sc=False,
                                         needs_layout_passes=False),
)(_tmf_body)


def _pack_bf16(t):
    t16 = t.astype(jnp.bfloat16)
    pairs = jnp.stack([t16[:, :L], t16[:, L:]], axis=-1)
    return jax.lax.bitcast_convert_type(pairs, jnp.uint8).reshape(-1, 4 * L)


def kernel(user_id, item_id, user_taggs, item_taggs,
           user_emb, item_emb, user_tagg_emb, item_tagg_emb):
    uid = user_id.astype(jnp.int32)
    iid = item_id.astype(jnp.int32)
    tag_idx = jnp.concatenate(
        [user_taggs.astype(jnp.int32), item_taggs.astype(jnp.int32) + NT],
        axis=1).reshape(-1)
    tbl = _pack_bf16(jnp.concatenate([user_tagg_emb, item_tagg_emb], axis=0))
    rows = jnp.concatenate(
        [jnp.take(user_emb, uid, axis=0), jnp.take(item_emb, iid, axis=0)],
        axis=1).reshape(-1)
    return _tmf(tag_idx, rows, tbl)


# speedup vs baseline: 1.6407x; 1.6407x over previous
import functools

import jax
import jax.numpy as jnp
from jax import lax
from jax.experimental import pallas as pl
from jax.experimental.pallas import tpu as pltpu
from jax.experimental.pallas import tpu_sc as plsc

D = 32
HIST = 50
HT = 2 * HIST
RW = 2 * D
NT = 100000
NC, NS, L = 2, 16, 16
NW = NC * NS
B = 16384
BT = B // NW
C = 16
NCH = BT // C
CI = C * HT


def _tmf_body(tag_h, rows_h, tbl_h, out_h,
              tag_i, tr, rw, out_v, sem0, sem1):
    wid = lax.axis_index("s") * NC + lax.axis_index("c")
    base = wid * BT

    pltpu.sync_copy(tag_h.at[pl.ds(base * HT, BT * HT)], tag_i)

    def issue(g, k, sem):
        pltpu.async_copy(tbl_h.at[tag_i.at[pl.ds(g * CI, CI)]], tr.at[k], sem)
        pltpu.async_copy(rows_h.at[pl.ds((base + g * C) * RW, C * RW)],
                         rw.at[k], sem)

    def drain(k, sem):
        pltpu.make_async_copy(tbl_h.at[pl.ds(0, CI)], tr.at[k], sem).wait()
        pltpu.make_async_copy(rows_h.at[pl.ds(0, C * RW)], rw.at[k], sem).wait()

    himask = jnp.uint32(0xFFFF0000)

    def _lo(w):
        return plsc.bitcast(w << jnp.uint32(16), jnp.float32)

    def _hi(w):
        return plsc.bitcast(w & himask, jnp.float32)

    def compute(g, k):
        def body_b(b, acc):
            r0 = b * HT
            wu = tr[k, r0, pl.ds(0, L)]
            wi = tr[k, r0 + HIST, pl.ds(0, L)]
            u0, u1 = _lo(wu), _hi(wu)
            i0, i1 = _lo(wi), _hi(wi)
            for h in range(1, HIST):
                wu = tr[k, r0 + h, pl.ds(0, L)]
                wi = tr[k, r0 + HIST + h, pl.ds(0, L)]
                u0 = u0 + _lo(wu)
                u1 = u1 + _hi(wu)
                i0 = i0 + _lo(wi)
                i1 = i1 + _hi(wi)
            inv = 1.0 / HIST
            ru0 = rw[k, pl.ds(b * RW, L)] + u0 * inv
            ru1 = rw[k, pl.ds(b * RW + L, L)] + u1 * inv
            ri0 = rw[k, pl.ds(b * RW + D, L)] + i0 * inv
            ri1 = rw[k, pl.ds(b * RW + D + L, L)] + i1 * inv
            s = ru0 * ri0 + ru1 * ri1
            iot = lax.iota(jnp.int32, L)
            for k2 in (8, 4, 2, 1):
                s = s + s[jnp.bitwise_xor(iot, k2)]
            return jnp.where(iot == b, s, acc)
        a = lax.fori_loop(0, C, body_b, jnp.zeros((L,), jnp.float32))
        out_v[pl.ds(g * C, C)] = a

    issue(0, 0, sem0)

    def pair(t, carry):
        g0 = 2 * t
        issue(g0 + 1, 1, sem1)
        drain(0, sem0)
        compute(g0, 0)

        @pl.when(t < NCH // 2 - 1)
        def _():
            issue(g0 + 2, 0, sem0)

        drain(1, sem1)
        compute(g0 + 1, 1)
        return carry

    lax.fori_loop(0, NCH // 2, pair, 0)

    pltpu.sync_copy(out_v, out_h.at[pl.ds(base, BT)])


_tmf = functools.partial(
    pl.kernel,
    out_type=jax.ShapeDtypeStruct((B,), jnp.float32),
    mesh=plsc.VectorSubcoreMesh(core_axis_name="c", subcore_axis_name="s",
                                num_cores=NC, num_subcores=NS),
    scratch_types=[
        pltpu.VMEM((BT * HT,), jnp.int32),
        pltpu.VMEM((2, CI, L), jnp.uint32),
        pltpu.VMEM((2, C * RW), jnp.float32),
        pltpu.VMEM((BT,), jnp.float32),
        pltpu.SemaphoreType.DMA,
        pltpu.SemaphoreType.DMA,
    ],
    compiler_params=pltpu.CompilerParams(use_tc_tiling_on_sc=False,
                                         needs_layout_passes=False),
)(_tmf_body)


def _pack_bf16(t):
    t16 = t.astype(jnp.bfloat16)
    pairs = jnp.stack([t16[:, :L], t16[:, L:]], axis=-1)
    return jax.lax.bitcast_convert_type(pairs, jnp.uint32)


def kernel(user_id, item_id, user_taggs, item_taggs,
           user_emb, item_emb, user_tagg_emb, item_tagg_emb):
    uid = user_id.astype(jnp.int32)
    iid = item_id.astype(jnp.int32)
    tag_idx = jnp.concatenate(
        [user_taggs.astype(jnp.int32), item_taggs.astype(jnp.int32) + NT],
        axis=1).reshape(-1)
    tbl = _pack_bf16(jnp.concatenate([user_tagg_emb, item_tagg_emb], axis=0))
    rows = jnp.concatenate(
        [jnp.take(user_emb, uid, axis=0), jnp.take(item_emb, iid, axis=0)],
        axis=1).reshape(-1)
    return _tmf(tag_idx, rows, tbl)
